# speedup vs baseline: 1.0742x; 1.0742x over previous
import jax
import jax.numpy as jnp
from jax.experimental import pallas as pl


def _pos_body(row_ref, col_ref, out_ref):
    h = out_ref.shape[1]
    w = out_ref.shape[2]
    f = col_ref.shape[1]
    col = col_ref[0:w, :]
    row = row_ref[0:h, :]
    x_part = jnp.broadcast_to(col[None, :, :], (h, w, f))
    y_part = jnp.broadcast_to(row[:, None, :], (h, w, f))
    out_ref[0] = jnp.concatenate([x_part, y_part], axis=-1)


def kernel(img, row_embed, col_embed):
    b, h, w = img.shape[0], img.shape[1], img.shape[2]
    f = col_embed.shape[1]
    out_shape = jax.ShapeDtypeStruct((b, h, w, 2 * f), col_embed.dtype)
    return pl.pallas_call(
        _pos_body,
        grid=(b,),
        in_specs=[
            pl.BlockSpec(row_embed.shape, lambda i: (0, 0)),
            pl.BlockSpec(col_embed.shape, lambda i: (0, 0)),
        ],
        out_specs=pl.BlockSpec((1, h, w, 2 * f), lambda i: (i, 0, 0, 0)),
        out_shape=out_shape,
    )(row_embed, col_embed)
